# baseline (speedup 1.0000x reference)
import functools

import jax
import jax.numpy as jnp
from jax import lax
from jax.experimental import pallas as pl
from jax.experimental.pallas import tpu as pltpu
from jax.experimental.pallas import tpu_sc as plsc

_TOKENS = 8192
_DIM = 2048
_NE = 64
_NG = 8
_TK = 8
_SCALE = 2.5
_NEG = float("-inf")
_NC = 2
_NS = 16

_SORT8 = [(0, 1), (2, 3), (4, 5), (6, 7), (0, 2), (1, 3), (4, 6), (5, 7),
          (1, 2), (5, 6), (0, 4), (1, 5), (2, 6), (3, 7), (2, 4), (3, 5),
          (1, 2), (3, 4), (5, 6)]


def _dense_body(x_ref, wt_ref, b_ref, rb_ref, s_ref):
    logits = jnp.dot(x_ref[...], wt_ref[...],
                     preferred_element_type=jnp.float32)
    sig = jax.nn.sigmoid(logits + b_ref[...])
    s = sig / jnp.sum(sig, axis=-1, keepdims=True) + rb_ref[...]
    gm = [jnp.max(s[:, 8 * g:8 * (g + 1)], axis=1, keepdims=True)
          for g in range(_NG)]
    srt = list(gm)
    for i, j in _SORT8:
        srt[i], srt[j] = (jnp.minimum(srt[i], srt[j]),
                          jnp.maximum(srt[i], srt[j]))
    thr = srt[4]
    s_ref[...] = jnp.concatenate(
        [jnp.where(gm[g] >= thr, s[:, 8 * g:8 * (g + 1)], _NEG)
         for g in range(_NG)], axis=1)


def _dense_scores(x, wt, b, rb):
    blk = 2048
    n = x.shape[0]
    return pl.pallas_call(
        _dense_body,
        grid=(n // blk,),
        in_specs=[
            pl.BlockSpec((blk, _DIM), lambda i: (i, 0)),
            pl.BlockSpec((_DIM, _NE), lambda i: (0, 0)),
            pl.BlockSpec((1, _NE), lambda i: (0, 0)),
            pl.BlockSpec((1, _NE), lambda i: (0, 0)),
        ],
        out_specs=pl.BlockSpec((blk, _NE), lambda i: (i, 0)),
        out_shape=jax.ShapeDtypeStruct((n, _NE), jnp.float32),
        compiler_params=pltpu.CompilerParams(
            dimension_semantics=("arbitrary",)),
    )(x, wt, b, rb)


def _routing(scores):
    nw = _NC * _NS
    ntok = scores.shape[0]
    tpw = ntok // nw
    mesh = plsc.VectorSubcoreMesh(core_axis_name="c", subcore_axis_name="s")

    @functools.partial(
        pl.kernel,
        mesh=mesh,
        out_type=[jax.ShapeDtypeStruct((ntok * _TK,), jnp.float32),
                  jax.ShapeDtypeStruct((ntok * _TK,), jnp.int32)],
        scratch_types=[pltpu.VMEM((tpw, _NE), jnp.float32),
                       pltpu.VMEM((tpw * _TK + 8,), jnp.float32),
                       pltpu.VMEM((tpw * _TK + 8,), jnp.int32),
                       pltpu.VMEM((12, 24), jnp.float32),
                       pltpu.VMEM((12, 24), jnp.int32)],
        compiler_params=pltpu.CompilerParams(needs_layout_passes=False),
    )
    def body(scores_hbm, vals_hbm, idx_hbm, sbuf, vbuf, ibuf, mk, mv):
        wid = lax.axis_index("s") * _NC + lax.axis_index("c")
        base = wid * tpw
        pltpu.sync_copy(scores_hbm.at[pl.ds(base, tpw)], sbuf)
        lane = lax.iota(jnp.int32, 16)
        eids = [lane + 16 * i for i in range(4)]

        def merge(slot, ka, va, kb, vb):
            mk[slot, pl.ds(0, 16)] = ka
            mk[slot, pl.ds(8, 16)] = kb
            mv[slot, pl.ds(0, 16)] = va
            mv[slot, pl.ds(8, 16)] = vb
            return plsc.sort_key_val(mk[slot, pl.ds(0, 16)],
                                     mv[slot, pl.ds(0, 16)],
                                     descending=True)

        def tok(g, carry):
            for u in range(4):
                t = 4 * g + u
                srt = [plsc.sort_key_val(sbuf[t, pl.ds(16 * i, 16)],
                                         eids[i], descending=True)
                       for i in range(4)]
                k01, v01 = merge(3 * u, *srt[0], *srt[1])
                k23, v23 = merge(3 * u + 1, *srt[2], *srt[3])
                kf, vf = merge(3 * u + 2, k01, v01, k23, v23)
                vbuf[pl.ds(_TK * t, 16)] = kf * _SCALE
                ibuf[pl.ds(_TK * t, 16)] = vf
            return carry

        lax.fori_loop(0, tpw // 4, tok, 0)
        pltpu.sync_copy(vbuf.at[pl.ds(0, tpw * _TK)],
                        vals_hbm.at[pl.ds(base * _TK, tpw * _TK)])
        pltpu.sync_copy(ibuf.at[pl.ds(0, tpw * _TK)],
                        idx_hbm.at[pl.ds(base * _TK, tpw * _TK)])

    return body(scores)


def kernel(x, w1_weight, w1_bias, router_bias):
    scores = _dense_scores(x, w1_weight.T, w1_bias.reshape(1, _NE),
                           router_bias.reshape(1, _NE))
    vals, ids = _routing(scores)
    return vals.reshape(_TOKENS, _TK), ids.reshape(_TOKENS, _TK)


# speedup vs baseline: 1.0123x; 1.0123x over previous
import functools

import jax
import jax.numpy as jnp
from jax import lax
from jax.experimental import pallas as pl
from jax.experimental.pallas import tpu as pltpu
from jax.experimental.pallas import tpu_sc as plsc

_TOKENS = 8192
_DIM = 2048
_NE = 64
_NG = 8
_TK = 8
_SCALE = 2.5
_NEG = float("-inf")
_NC = 2
_NS = 16

_SORT8 = [(0, 1), (2, 3), (4, 5), (6, 7), (0, 2), (1, 3), (4, 6), (5, 7),
          (1, 2), (5, 6), (0, 4), (1, 5), (2, 6), (3, 7), (2, 4), (3, 5),
          (1, 2), (3, 4), (5, 6)]


def _dense_body(x_ref, wt_ref, b_ref, rb_ref, s_ref):
    logits = jnp.dot(x_ref[...], wt_ref[...],
                     preferred_element_type=jnp.float32)
    sig = jax.nn.sigmoid(logits + b_ref[...])
    s = sig / jnp.sum(sig, axis=-1, keepdims=True) + rb_ref[...]
    gm = [jnp.max(s[:, 8 * g:8 * (g + 1)], axis=1, keepdims=True)
          for g in range(_NG)]
    srt = list(gm)
    for i, j in _SORT8:
        srt[i], srt[j] = (jnp.minimum(srt[i], srt[j]),
                          jnp.maximum(srt[i], srt[j]))
    thr = srt[4]
    s_ref[...] = jnp.concatenate(
        [jnp.where(gm[g] >= thr, s[:, 8 * g:8 * (g + 1)], _NEG)
         for g in range(_NG)], axis=1)


def _dense_scores(x, wt, b, rb):
    blk = 2048
    n = x.shape[0]
    return pl.pallas_call(
        _dense_body,
        grid=(n // blk,),
        in_specs=[
            pl.BlockSpec((blk, _DIM), lambda i: (i, 0)),
            pl.BlockSpec((_DIM, _NE), lambda i: (0, 0)),
            pl.BlockSpec((1, _NE), lambda i: (0, 0)),
            pl.BlockSpec((1, _NE), lambda i: (0, 0)),
        ],
        out_specs=pl.BlockSpec((blk, _NE), lambda i: (i, 0)),
        out_shape=jax.ShapeDtypeStruct((n, _NE), jnp.float32),
        compiler_params=pltpu.CompilerParams(
            dimension_semantics=("arbitrary",)),
    )(x, wt, b, rb)


def _routing(scores):
    nw = _NC * _NS
    ntok = scores.shape[0]
    tpw = ntok // nw
    mesh = plsc.VectorSubcoreMesh(core_axis_name="c", subcore_axis_name="s")

    @functools.partial(
        pl.kernel,
        mesh=mesh,
        out_type=[jax.ShapeDtypeStruct((ntok * _TK,), jnp.float32),
                  jax.ShapeDtypeStruct((ntok * _TK,), jnp.int32)],
        scratch_types=[pltpu.VMEM((2, tpw // 2, _NE), jnp.float32),
                       pltpu.VMEM((tpw * _TK + 8,), jnp.float32),
                       pltpu.VMEM((tpw * _TK + 8,), jnp.int32),
                       pltpu.SemaphoreType.DMA,
                       pltpu.SemaphoreType.DMA],
        compiler_params=pltpu.CompilerParams(needs_layout_passes=False),
    )
    def body(scores_hbm, vals_hbm, idx_hbm, sbuf, vbuf, ibuf, sem0, sem1):
        wid = lax.axis_index("s") * _NC + lax.axis_index("c")
        base = wid * tpw
        seg = tpw // 2
        lane = lax.iota(jnp.int32, 16)
        eids = [lane + 16 * i for i in range(4)]

        def bmerge(ka, va, kb, vb):
            kr = lax.rev(kb, (0,))
            vr = lax.rev(vb, (0,))
            sel = ka >= kr
            return plsc.sort_key_val(jnp.where(sel, ka, kr),
                                     jnp.where(sel, va, vr),
                                     descending=True)

        copies = [
            pltpu.async_copy(scores_hbm.at[pl.ds(base, seg)],
                             sbuf.at[0], sem0),
            pltpu.async_copy(scores_hbm.at[pl.ds(base + seg, seg)],
                             sbuf.at[1], sem1),
        ]

        for s in range(2):
            copies[s].wait()

            def tok(g, carry, s=s):
                for u in range(4):
                    t = 4 * g + u
                    srt = [plsc.sort_key_val(sbuf[s, t, pl.ds(16 * i, 16)],
                                             eids[i], descending=True)
                           for i in range(4)]
                    kf, vf = bmerge(*bmerge(*srt[0], *srt[1]),
                                    *bmerge(*srt[2], *srt[3]))
                    o = _TK * (s * seg + 4 * g + u)
                    vbuf[pl.ds(o, 16)] = kf * _SCALE
                    ibuf[pl.ds(o, 16)] = vf
                return carry

            lax.fori_loop(0, seg // 4, tok, 0)

        pltpu.sync_copy(vbuf.at[pl.ds(0, tpw * _TK)],
                        vals_hbm.at[pl.ds(base * _TK, tpw * _TK)])
        pltpu.sync_copy(ibuf.at[pl.ds(0, tpw * _TK)],
                        idx_hbm.at[pl.ds(base * _TK, tpw * _TK)])

    return body(scores)


def kernel(x, w1_weight, w1_bias, router_bias):
    scores = _dense_scores(x, w1_weight.T, w1_bias.reshape(1, _NE),
                           router_bias.reshape(1, _NE))
    vals, ids = _routing(scores)
    return vals.reshape(_TOKENS, _TK), ids.reshape(_TOKENS, _TK)


# speedup vs baseline: 1.0193x; 1.0069x over previous
import functools

import jax
import jax.numpy as jnp
from jax import lax
from jax.experimental import pallas as pl
from jax.experimental.pallas import tpu as pltpu
from jax.experimental.pallas import tpu_sc as plsc

_TOKENS = 8192
_DIM = 2048
_NE = 64
_NG = 8
_TK = 8
_SCALE = 2.5
_NEG = float("-inf")
_NC = 2
_NS = 16

_SORT8 = [(0, 1), (2, 3), (4, 5), (6, 7), (0, 2), (1, 3), (4, 6), (5, 7),
          (1, 2), (5, 6), (0, 4), (1, 5), (2, 6), (3, 7), (2, 4), (3, 5),
          (1, 2), (3, 4), (5, 6)]


def _dense_body(x_ref, wt_ref, b_ref, rb_ref, s_ref):
    logits = jnp.dot(x_ref[...], wt_ref[...],
                     preferred_element_type=jnp.float32)
    sig = jax.nn.sigmoid(logits + b_ref[...])
    s = sig / jnp.sum(sig, axis=-1, keepdims=True) + rb_ref[...]
    gm = [jnp.max(s[:, 8 * g:8 * (g + 1)], axis=1, keepdims=True)
          for g in range(_NG)]
    srt = list(gm)
    for i, j in _SORT8:
        srt[i], srt[j] = (jnp.minimum(srt[i], srt[j]),
                          jnp.maximum(srt[i], srt[j]))
    thr = srt[4]
    s_ref[...] = jnp.concatenate(
        [jnp.where(gm[g] >= thr, s[:, 8 * g:8 * (g + 1)], _NEG)
         for g in range(_NG)], axis=1)


def _dense_scores(x, wt, b, rb):
    blk = 1024
    n = x.shape[0]
    return pl.pallas_call(
        _dense_body,
        grid=(n // blk,),
        in_specs=[
            pl.BlockSpec((blk, _DIM), lambda i: (i, 0)),
            pl.BlockSpec((_DIM, _NE), lambda i: (0, 0)),
            pl.BlockSpec((1, _NE), lambda i: (0, 0)),
            pl.BlockSpec((1, _NE), lambda i: (0, 0)),
        ],
        out_specs=pl.BlockSpec((blk, _NE), lambda i: (i, 0)),
        out_shape=jax.ShapeDtypeStruct((n, _NE), jnp.float32),
        compiler_params=pltpu.CompilerParams(
            dimension_semantics=("arbitrary",)),
    )(x, wt, b, rb)


def _routing(scores):
    nw = _NC * _NS
    ntok = scores.shape[0]
    tpw = ntok // nw
    mesh = plsc.VectorSubcoreMesh(core_axis_name="c", subcore_axis_name="s")

    @functools.partial(
        pl.kernel,
        mesh=mesh,
        out_type=[jax.ShapeDtypeStruct((ntok * _TK,), jnp.float32),
                  jax.ShapeDtypeStruct((ntok * _TK,), jnp.int32)],
        scratch_types=[pltpu.VMEM((2, tpw // 2, _NE), jnp.float32),
                       pltpu.VMEM((tpw * _TK + 8,), jnp.float32),
                       pltpu.VMEM((tpw * _TK + 8,), jnp.int32),
                       pltpu.SemaphoreType.DMA,
                       pltpu.SemaphoreType.DMA],
        compiler_params=pltpu.CompilerParams(needs_layout_passes=False),
    )
    def body(scores_hbm, vals_hbm, idx_hbm, sbuf, vbuf, ibuf, sem0, sem1):
        wid = lax.axis_index("s") * _NC + lax.axis_index("c")
        base = wid * tpw
        seg = tpw // 2
        lane = lax.iota(jnp.int32, 16)
        eids = [lane + 16 * i for i in range(4)]

        def bmerge(ka, va, kb, vb):
            kr = lax.rev(kb, (0,))
            vr = lax.rev(vb, (0,))
            sel = ka >= kr
            return plsc.sort_key_val(jnp.where(sel, ka, kr),
                                     jnp.where(sel, va, vr),
                                     descending=True)

        copies = [
            pltpu.async_copy(scores_hbm.at[pl.ds(base, seg)],
                             sbuf.at[0], sem0),
            pltpu.async_copy(scores_hbm.at[pl.ds(base + seg, seg)],
                             sbuf.at[1], sem1),
        ]

        for s in range(2):
            copies[s].wait()

            def tok(g, carry, s=s):
                for u in range(4):
                    t = 4 * g + u
                    srt = [plsc.sort_key_val(sbuf[s, t, pl.ds(16 * i, 16)],
                                             eids[i], descending=True)
                           for i in range(4)]
                    kf, vf = bmerge(*bmerge(*srt[0], *srt[1]),
                                    *bmerge(*srt[2], *srt[3]))
                    o = _TK * (s * seg + 4 * g + u)
                    vbuf[pl.ds(o, 16)] = kf * _SCALE
                    ibuf[pl.ds(o, 16)] = vf
                return carry

            lax.fori_loop(0, seg // 4, tok, 0)

        pltpu.sync_copy(vbuf.at[pl.ds(0, tpw * _TK)],
                        vals_hbm.at[pl.ds(base * _TK, tpw * _TK)])
        pltpu.sync_copy(ibuf.at[pl.ds(0, tpw * _TK)],
                        idx_hbm.at[pl.ds(base * _TK, tpw * _TK)])

    return body(scores)


def kernel(x, w1_weight, w1_bias, router_bias):
    scores = _dense_scores(x, w1_weight.T, w1_bias.reshape(1, _NE),
                           router_bias.reshape(1, _NE))
    vals, ids = _routing(scores)
    return vals.reshape(_TOKENS, _TK), ids.reshape(_TOKENS, _TK)
